# Initial kernel scaffold; baseline (speedup 1.0000x reference)
import jax
import jax.numpy as jnp
from jax.experimental import pallas as pl


def kernel(inputs, embeddings):
    raise NotImplementedError("write your pallas kernel here")


# speedup vs baseline: 1.3243x; 1.3243x over previous
import functools

import jax
import jax.numpy as jnp
from jax import lax
from jax.experimental import pallas as pl
from jax.experimental.pallas import tpu as pltpu
from jax.experimental.pallas import tpu_sc as plsc

_K = 8192
_CB = 4
_SUB = 64
_OUT = 256
_RBLK = 512

_NC = 2
_NS = 16
_NW = _NC * _NS
_CH = 128


def _dist_argmin_body(x_ref, x2_ref, e_ref, e2_ref, idx_ref, min_ref):
    for k in range(_CB):
        x = x_ref[:, k * _SUB:(k + 1) * _SUB].astype(jnp.bfloat16)
        e = e_ref[k].astype(jnp.bfloat16)
        m = lax.dot_general(x, e, (((1,), (1,)), ((), ())),
                            preferred_element_type=jnp.float32)
        d = (x2_ref[:, k:k + 1] + e2_ref[k]) - 2.0 * m
        mv = jnp.min(d, axis=1, keepdims=True)
        lanes = lax.broadcasted_iota(jnp.int32, d.shape, 1)
        idx = jnp.min(jnp.where(d == mv, lanes, _K), axis=1, keepdims=True)
        idx_ref[:, k:k + 1] = idx
        min_ref[:, k:k + 1] = mv


def _dist_argmin(flat, x2, embeddings, e2, n_rows):
    nrb = n_rows // _RBLK
    return pl.pallas_call(
        _dist_argmin_body,
        grid=(nrb,),
        in_specs=[
            pl.BlockSpec((_RBLK, _OUT), lambda rb: (rb, 0)),
            pl.BlockSpec((_RBLK, _CB), lambda rb: (rb, 0)),
            pl.BlockSpec((_CB, _K, _SUB), lambda rb: (0, 0, 0)),
            pl.BlockSpec((_CB, 1, _K), lambda rb: (0, 0, 0)),
        ],
        out_specs=[
            pl.BlockSpec((_RBLK, _CB), lambda rb: (rb, 0)),
            pl.BlockSpec((_RBLK, _CB), lambda rb: (rb, 0)),
        ],
        out_shape=[
            jax.ShapeDtypeStruct((n_rows, _CB), jnp.int32),
            jax.ShapeDtypeStruct((n_rows, _CB), jnp.float32),
        ],
        compiler_params=pltpu.CompilerParams(
            dimension_semantics=("arbitrary",),
        ),
    )(flat, x2, embeddings, e2)


def _sc_gather(table, idx3, total_rows):
    nch = idx3.shape[1]
    b_per_w = nch * _CH
    mesh = plsc.VectorSubcoreMesh(core_axis_name="c", subcore_axis_name="s")

    @functools.partial(
        pl.kernel,
        mesh=mesh,
        out_type=jax.ShapeDtypeStruct((total_rows, _SUB), jnp.float32),
        compiler_params=pltpu.CompilerParams(use_tc_tiling_on_sc=False),
        scratch_types=[
            pltpu.VMEM((nch, _CH), jnp.int32),
            pltpu.VMEM((_CH, _SUB), jnp.float32),
            pltpu.VMEM((_CH, _SUB), jnp.float32),
            pltpu.SemaphoreType.DMA,
            pltpu.SemaphoreType.DMA,
        ],
    )
    def gather_kernel(table_hbm, idx_hbm, out_hbm, idx_v, rows_a, rows_b, sem_a, sem_b):
        wid = lax.axis_index("s") * _NC + lax.axis_index("c")
        base = wid * b_per_w
        pltpu.sync_copy(idx_hbm.at[wid], idx_v)
        bufs = (rows_a, rows_b)
        sems = (sem_a, sem_b)
        copies = [None, None]
        for j in range(nch):
            b = j % 2
            if copies[b] is not None:
                copies[b].wait()
                pltpu.sync_copy(bufs[b], out_hbm.at[pl.ds(base + (j - 2) * _CH, _CH)])
            copies[b] = pltpu.async_copy(table_hbm.at[idx_v.at[j]], bufs[b], sems[b])
        for j in (nch - 2, nch - 1):
            b = j % 2
            copies[b].wait()
            pltpu.sync_copy(bufs[b], out_hbm.at[pl.ds(base + j * _CH, _CH)])

    return gather_kernel(table, idx3)


def kernel(inputs, embeddings):
    in_shape = inputs.shape
    flat = inputs.reshape(-1, _OUT)
    n = flat.shape[0]
    chunks = jnp.split(flat, _CB, axis=1)
    x2 = jnp.concatenate(
        [jnp.sum(c ** 2, axis=1, keepdims=True) for c in chunks], axis=1)
    e2 = jnp.stack([jnp.sum(embeddings[k] ** 2, axis=1) for k in range(_CB)])
    e2 = e2[:, None, :]

    idx_t, min_t = _dist_argmin(flat, x2, embeddings, e2, n)

    table = embeddings.reshape(_CB * _K, _SUB)
    gidx = idx_t + (jnp.arange(_CB, dtype=jnp.int32) * _K)[None, :]
    idx3d = gidx.reshape(_NW, -1, _CH)
    rows = _sc_gather(table, idx3d, _CB * n)
    quantized = rows.reshape(n, _OUT)

    quantized_sg = quantized.reshape(in_shape)
    loss = (1.25 * jnp.sum(min_t, axis=1)).reshape(in_shape[:-1])
    enc = jnp.zeros((n, 1), jnp.int32)
    for k in range(_CB):
        enc = enc * _K + idx_t[:, k:k + 1]
    return (quantized_sg, enc, loss)


# speedup vs baseline: 1.3248x; 1.0003x over previous
import functools

import jax
import jax.numpy as jnp
from jax import lax
from jax.experimental import pallas as pl
from jax.experimental.pallas import tpu as pltpu
from jax.experimental.pallas import tpu_sc as plsc

_K = 8192
_CB = 4
_SUB = 64
_OUT = 256
_RBLK = 512

_NC = 2
_NS = 16
_NW = _NC * _NS
_CH = 128
_NBUF = 6


def _dist_argmin_body(x_ref, x2_ref, e_ref, e2_ref, idx_ref, min_ref):
    for k in range(_CB):
        x = x_ref[:, k * _SUB:(k + 1) * _SUB].astype(jnp.bfloat16)
        e = e_ref[k].astype(jnp.bfloat16)
        m = lax.dot_general(x, e, (((1,), (1,)), ((), ())),
                            preferred_element_type=jnp.float32)
        d = (x2_ref[:, k:k + 1] + e2_ref[k]) - 2.0 * m
        mv = jnp.min(d, axis=1, keepdims=True)
        lanes = lax.broadcasted_iota(jnp.int32, d.shape, 1)
        idx = jnp.min(jnp.where(d == mv, lanes, _K), axis=1, keepdims=True)
        idx_ref[:, k:k + 1] = idx
        min_ref[:, k:k + 1] = mv


def _dist_argmin(flat, x2, embeddings, e2, n_rows):
    nrb = n_rows // _RBLK
    return pl.pallas_call(
        _dist_argmin_body,
        grid=(nrb,),
        in_specs=[
            pl.BlockSpec((_RBLK, _OUT), lambda rb: (rb, 0)),
            pl.BlockSpec((_RBLK, _CB), lambda rb: (rb, 0)),
            pl.BlockSpec((_CB, _K, _SUB), lambda rb: (0, 0, 0)),
            pl.BlockSpec((_CB, 1, _K), lambda rb: (0, 0, 0)),
        ],
        out_specs=[
            pl.BlockSpec((_RBLK, _CB), lambda rb: (rb, 0)),
            pl.BlockSpec((_RBLK, _CB), lambda rb: (rb, 0)),
        ],
        out_shape=[
            jax.ShapeDtypeStruct((n_rows, _CB), jnp.int32),
            jax.ShapeDtypeStruct((n_rows, _CB), jnp.float32),
        ],
        compiler_params=pltpu.CompilerParams(
            dimension_semantics=("arbitrary",),
        ),
    )(flat, x2, embeddings, e2)


def _sc_gather(table, idx3, total_rows):
    nch = idx3.shape[1]
    b_per_w = nch * _CH
    mesh = plsc.VectorSubcoreMesh(core_axis_name="c", subcore_axis_name="s")

    @functools.partial(
        pl.kernel,
        mesh=mesh,
        out_type=jax.ShapeDtypeStruct((total_rows, _SUB), jnp.float32),
        compiler_params=pltpu.CompilerParams(use_tc_tiling_on_sc=False),
        scratch_types=(
            [pltpu.VMEM((nch, _CH), jnp.int32)]
            + [pltpu.VMEM((_CH, _SUB), jnp.float32) for _ in range(_NBUF)]
            + [pltpu.SemaphoreType.DMA for _ in range(_NBUF)]
        ),
    )
    def gather_kernel(table_hbm, idx_hbm, out_hbm, idx_v, *bufsem):
        bufs = bufsem[:_NBUF]
        sems = bufsem[_NBUF:]
        wid = lax.axis_index("s") * _NC + lax.axis_index("c")
        base = wid * b_per_w
        pltpu.sync_copy(idx_hbm.at[wid], idx_v)
        copies = [None] * _NBUF
        for j in range(nch):
            b = j % _NBUF
            if copies[b] is not None:
                copies[b].wait()
                pltpu.sync_copy(bufs[b], out_hbm.at[pl.ds(base + (j - _NBUF) * _CH, _CH)])
            copies[b] = pltpu.async_copy(table_hbm.at[idx_v.at[j]], bufs[b], sems[b])
        for j in range(max(0, nch - _NBUF), nch):
            b = j % _NBUF
            copies[b].wait()
            pltpu.sync_copy(bufs[b], out_hbm.at[pl.ds(base + j * _CH, _CH)])

    return gather_kernel(table, idx3)


def kernel(inputs, embeddings):
    in_shape = inputs.shape
    flat = inputs.reshape(-1, _OUT)
    n = flat.shape[0]
    chunks = jnp.split(flat, _CB, axis=1)
    x2 = jnp.concatenate(
        [jnp.sum(c ** 2, axis=1, keepdims=True) for c in chunks], axis=1)
    e2 = jnp.stack([jnp.sum(embeddings[k] ** 2, axis=1) for k in range(_CB)])
    e2 = e2[:, None, :]

    idx_t, min_t = _dist_argmin(flat, x2, embeddings, e2, n)

    table = embeddings.reshape(_CB * _K, _SUB)
    gidx = idx_t + (jnp.arange(_CB, dtype=jnp.int32) * _K)[None, :]
    idx3d = gidx.reshape(_NW, -1, _CH)
    rows = _sc_gather(table, idx3d, _CB * n)
    quantized = rows.reshape(n, _OUT)

    quantized_sg = quantized.reshape(in_shape)
    loss = (1.25 * jnp.sum(min_t, axis=1)).reshape(in_shape[:-1])
    enc = jnp.zeros((n, 1), jnp.int32)
    for k in range(_CB):
        enc = enc * _K + idx_t[:, k:k + 1]
    return (quantized_sg, enc, loss)
